# baseline (device time: 100183 ns/iter reference)
import jax
import jax.numpy as jnp
from jax import lax
from jax.experimental import pallas as pl
from jax.experimental.pallas import tpu as pltpu

N_DEV = 8
T = 512
D = 256
H = 512
E = 16
E_LOC = 2


def kernel(x, router_W, route_idx, expert_W):
    def body(x_ref, rw_ref, idx_ref, ew_ref, out_ref,
             comm_ref, send_sems, recv_sems):
        my = lax.axis_index("i")
        left = lax.rem(my + N_DEV - 1, N_DEV)
        right = lax.rem(my + 1, N_DEV)

        barrier_sem = pltpu.get_barrier_semaphore()
        for nbr in (left, right):
            pl.semaphore_signal(
                barrier_sem, inc=1,
                device_id=(nbr,), device_id_type=pl.DeviceIdType.MESH,
            )
        pl.semaphore_wait(barrier_sem, 2)

        comm_ref[0] = ew_ref[...]

        xv = x_ref[...]
        scores = jnp.dot(xv, rw_ref[...], preferred_element_type=jnp.float32)
        s_max = jnp.max(scores, axis=1, keepdims=True)
        probs = jnp.exp(scores - s_max)
        probs = probs / jnp.sum(probs, axis=1, keepdims=True)
        e0 = idx_ref[:, 0:1]
        e1 = idx_ref[:, 1:2]
        iota = lax.broadcasted_iota(jnp.int32, (T, E), 1)
        sel = (iota == e0) | (iota == e1)
        wm = jnp.where(sel, probs, 0.0)
        wnorm = wm / jnp.sum(wm, axis=1, keepdims=True)

        acc = jnp.zeros((T, H), jnp.float32)
        for h in range(N_DEV):
            if h < N_DEV - 1:
                rdma = pltpu.make_async_remote_copy(
                    src_ref=comm_ref.at[h],
                    dst_ref=comm_ref.at[h + 1],
                    send_sem=send_sems.at[h],
                    recv_sem=recv_sems.at[h],
                    device_id=(right,),
                    device_id_type=pl.DeviceIdType.MESH,
                )
                rdma.start()

            src = lax.rem(my + N_DEV - h, N_DEV)
            ea = 2 * src
            wa = jnp.sum(jnp.where(iota == ea, wnorm, 0.0),
                         axis=1, keepdims=True)
            wb = jnp.sum(jnp.where(iota == ea + 1, wnorm, 0.0),
                         axis=1, keepdims=True)
            acc = acc + jnp.dot(xv * wa, comm_ref[h, 0],
                                preferred_element_type=jnp.float32)
            acc = acc + jnp.dot(xv * wb, comm_ref[h, 1],
                                preferred_element_type=jnp.float32)

            if h < N_DEV - 1:
                rdma.wait()

        out_ref[...] = acc

    return pl.pallas_call(
        body,
        out_shape=jax.ShapeDtypeStruct((T, H), jnp.float32),
        in_specs=[
            pl.BlockSpec(memory_space=pltpu.VMEM),
            pl.BlockSpec(memory_space=pltpu.VMEM),
            pl.BlockSpec(memory_space=pltpu.VMEM),
            pl.BlockSpec(memory_space=pltpu.VMEM),
        ],
        out_specs=pl.BlockSpec(memory_space=pltpu.VMEM),
        scratch_shapes=[
            pltpu.VMEM((N_DEV, E_LOC, D, H), jnp.float32),
            pltpu.SemaphoreType.DMA((N_DEV - 1,)),
            pltpu.SemaphoreType.DMA((N_DEV - 1,)),
        ],
        compiler_params=pltpu.CompilerParams(collective_id=0),
    )(x, router_W, route_idx, expert_W)


# device time: 76120 ns/iter; 1.3161x vs baseline; 1.3161x over previous
import jax
import jax.numpy as jnp
from jax import lax
from jax.experimental import pallas as pl
from jax.experimental.pallas import tpu as pltpu

N_DEV = 8
T = 512
D = 256
H = 512
E = 16
E_LOC = 2


def kernel(x, router_W, route_idx, expert_W):
    def body(x_ref, rw_ref, idx_ref, ew_ref, out_ref,
             comm_ref, send_sems, recv_sems):
        my = lax.axis_index("i")

        barrier_sem = pltpu.get_barrier_semaphore()
        for k in range(1, N_DEV):
            pl.semaphore_signal(
                barrier_sem, inc=1,
                device_id=(lax.rem(my + k, N_DEV),),
                device_id_type=pl.DeviceIdType.MESH,
            )
        pl.semaphore_wait(barrier_sem, N_DEV - 1)

        rdmas = []
        for k in range(1, N_DEV):
            rdma = pltpu.make_async_remote_copy(
                src_ref=ew_ref,
                dst_ref=comm_ref.at[k],
                send_sem=send_sems.at[k],
                recv_sem=recv_sems.at[k],
                device_id=(lax.rem(my + k, N_DEV),),
                device_id_type=pl.DeviceIdType.MESH,
            )
            rdma.start()
            rdmas.append(rdma)

        xv = x_ref[...]
        scores = jnp.dot(xv, rw_ref[...], preferred_element_type=jnp.float32)
        s_max = jnp.max(scores, axis=1, keepdims=True)
        probs = jnp.exp(scores - s_max)
        probs = probs / jnp.sum(probs, axis=1, keepdims=True)
        e0 = idx_ref[:, 0:1]
        e1 = idx_ref[:, 1:2]
        iota = lax.broadcasted_iota(jnp.int32, (T, E), 1)
        sel = (iota == e0) | (iota == e1)
        wm = jnp.where(sel, probs, 0.0)
        wnorm = wm / jnp.sum(wm, axis=1, keepdims=True)

        def contribution(acc, chunk_ref, src):
            ea = 2 * src
            wa = jnp.sum(jnp.where(iota == ea, wnorm, 0.0),
                         axis=1, keepdims=True)
            wb = jnp.sum(jnp.where(iota == ea + 1, wnorm, 0.0),
                         axis=1, keepdims=True)
            acc = acc + jnp.dot(xv * wa, chunk_ref[0],
                                preferred_element_type=jnp.float32)
            acc = acc + jnp.dot(xv * wb, chunk_ref[1],
                                preferred_element_type=jnp.float32)
            return acc

        acc = jnp.zeros((T, H), jnp.float32)
        acc = contribution(acc, ew_ref, my)

        for k in range(1, N_DEV):
            rdmas[k - 1].wait_recv()
            acc = contribution(acc, comm_ref.at[k],
                               lax.rem(my + N_DEV - k, N_DEV))

        for rdma in rdmas:
            rdma.wait_send()

        out_ref[...] = acc

    return pl.pallas_call(
        body,
        out_shape=jax.ShapeDtypeStruct((T, H), jnp.float32),
        in_specs=[
            pl.BlockSpec(memory_space=pltpu.VMEM),
            pl.BlockSpec(memory_space=pltpu.VMEM),
            pl.BlockSpec(memory_space=pltpu.VMEM),
            pl.BlockSpec(memory_space=pltpu.VMEM),
        ],
        out_specs=pl.BlockSpec(memory_space=pltpu.VMEM),
        scratch_shapes=[
            pltpu.VMEM((N_DEV, E_LOC, D, H), jnp.float32),
            pltpu.SemaphoreType.DMA((N_DEV,)),
            pltpu.SemaphoreType.DMA((N_DEV,)),
        ],
        compiler_params=pltpu.CompilerParams(collective_id=0),
    )(x, router_W, route_idx, expert_W)


# device time: 42792 ns/iter; 2.3412x vs baseline; 1.7788x over previous
import jax
import jax.numpy as jnp
from jax import lax
from jax.experimental import pallas as pl
from jax.experimental.pallas import tpu as pltpu

N_DEV = 8
T = 512
D = 256
H = 512
E = 16
E_LOC = 2


def kernel(x, router_W, route_idx, expert_W):
    def body(x_ref, rw_ref, idx_ref, ew_ref, out_ref,
             comm_ref, send_buf, send_sems, recv_sems):
        my = lax.axis_index("i")

        send_buf[...] = ew_ref[...].astype(jnp.bfloat16)

        barrier_sem = pltpu.get_barrier_semaphore()
        for k in range(1, N_DEV):
            pl.semaphore_signal(
                barrier_sem, inc=1,
                device_id=(lax.rem(my + k, N_DEV),),
                device_id_type=pl.DeviceIdType.MESH,
            )
        pl.semaphore_wait(barrier_sem, N_DEV - 1)

        rdmas = []
        for k in range(1, N_DEV):
            rdma = pltpu.make_async_remote_copy(
                src_ref=send_buf,
                dst_ref=comm_ref.at[k],
                send_sem=send_sems.at[k],
                recv_sem=recv_sems.at[k],
                device_id=(lax.rem(my + k, N_DEV),),
                device_id_type=pl.DeviceIdType.MESH,
            )
            rdma.start()
            rdmas.append(rdma)

        xv = x_ref[...]
        scores = jnp.dot(xv, rw_ref[...], preferred_element_type=jnp.float32)
        s_max = jnp.max(scores, axis=1, keepdims=True)
        probs = jnp.exp(scores - s_max)
        probs = probs / jnp.sum(probs, axis=1, keepdims=True)
        e0 = idx_ref[:, 0:1]
        e1 = idx_ref[:, 1:2]
        iota = lax.broadcasted_iota(jnp.int32, (T, E), 1)
        sel = (iota == e0) | (iota == e1)
        wm = jnp.where(sel, probs, 0.0)
        wnorm = wm / jnp.sum(wm, axis=1, keepdims=True)

        def contribution(acc, chunk_ref, src, cast):
            ea = 2 * src
            wa = jnp.sum(jnp.where(iota == ea, wnorm, 0.0),
                         axis=1, keepdims=True)
            wb = jnp.sum(jnp.where(iota == ea + 1, wnorm, 0.0),
                         axis=1, keepdims=True)
            xa, xb = xv * wa, xv * wb
            if cast:
                xa, xb = xa.astype(jnp.bfloat16), xb.astype(jnp.bfloat16)
            acc = acc + jnp.dot(xa, chunk_ref[0],
                                preferred_element_type=jnp.float32)
            acc = acc + jnp.dot(xb, chunk_ref[1],
                                preferred_element_type=jnp.float32)
            return acc

        acc = jnp.zeros((T, H), jnp.float32)
        acc = contribution(acc, ew_ref, my, cast=False)

        for k in range(1, N_DEV):
            rdmas[k - 1].wait_recv()
            acc = contribution(acc, comm_ref.at[k],
                               lax.rem(my + N_DEV - k, N_DEV), cast=True)

        for rdma in rdmas:
            rdma.wait_send()

        out_ref[...] = acc

    return pl.pallas_call(
        body,
        out_shape=jax.ShapeDtypeStruct((T, H), jnp.float32),
        in_specs=[
            pl.BlockSpec(memory_space=pltpu.VMEM),
            pl.BlockSpec(memory_space=pltpu.VMEM),
            pl.BlockSpec(memory_space=pltpu.VMEM),
            pl.BlockSpec(memory_space=pltpu.VMEM),
        ],
        out_specs=pl.BlockSpec(memory_space=pltpu.VMEM),
        scratch_shapes=[
            pltpu.VMEM((N_DEV, E_LOC, D, H), jnp.bfloat16),
            pltpu.VMEM((E_LOC, D, H), jnp.bfloat16),
            pltpu.SemaphoreType.DMA((N_DEV,)),
            pltpu.SemaphoreType.DMA((N_DEV,)),
        ],
        compiler_params=pltpu.CompilerParams(collective_id=0),
    )(x, router_W, route_idx, expert_W)
